# baseline (speedup 1.0000x reference)
import functools

import jax
import jax.numpy as jnp
from jax import lax
from jax.experimental import pallas as pl
from jax.experimental.pallas import tpu as pltpu
from jax.experimental.pallas import tpu_sc as plsc

B, K, P, C, A = 16, 900, 900, 92, 117
NCLS = 80
KP = 904
HW = 456
PW = 2 * HW
CHUNKS = 29
BB = 8
PT = 300

R_H, R_O, R_CX, R_CY, R_W, R_HB, R_SC, R_LB = range(8)


def _score_label_body(lgt_ref, hido_ref, pbox_ref, out_ref):
    x = lgt_ref[...]
    cl = lax.broadcasted_iota(jnp.int32, (C, BB, K), 0)
    neg = jnp.float32(-jnp.inf)
    x80 = jnp.where(cl < NCLS, x, neg)
    m80 = jnp.max(x80, axis=0, keepdims=True)
    lab = jnp.min(jnp.where(x80 == m80, cl, C), axis=0)
    x81 = jnp.where(cl < NCLS + 1, x, neg)
    m81 = jnp.max(x81, axis=0, keepdims=True)
    se = jnp.sum(jnp.exp(x81 - m81), axis=0)
    score = jnp.exp(m80[0] - m81[0]) / se
    out_ref[:, pl.ds(0, 2), pl.ds(0, K)] = hido_ref[...]
    out_ref[:, pl.ds(2, 4), pl.ds(0, K)] = lax.bitcast_convert_type(
        pbox_ref[...], jnp.int32
    )
    out_ref[:, pl.ds(R_SC, 1), pl.ds(0, K)] = lax.bitcast_convert_type(
        score, jnp.int32
    )[:, None]
    out_ref[:, pl.ds(R_LB, 1), pl.ds(0, K)] = lab[:, None]


_score_label = pl.pallas_call(
    _score_label_body,
    grid=(B // BB,),
    in_specs=[
        pl.BlockSpec((C, BB, K), lambda i: (0, i, 0)),
        pl.BlockSpec((BB, 2, K), lambda i: (i, 0, 0)),
        pl.BlockSpec((BB, 4, K), lambda i: (i, 0, 0)),
    ],
    out_specs=pl.BlockSpec((BB, 8, KP), lambda i: (i, 0, 0)),
    out_shape=jax.ShapeDtypeStruct((B, 8, KP), jnp.int32),
)


_mesh = plsc.VectorSubcoreMesh(core_axis_name="c", subcore_axis_name="s")
_sc_params = pltpu.CompilerParams(
    needs_layout_passes=False, use_tc_tiling_on_sc=False
)


@functools.partial(
    pl.kernel,
    out_type=jax.ShapeDtypeStruct((B, PW), jnp.float32),
    mesh=_mesh,
    compiler_params=_sc_params,
    scratch_types=[
        pltpu.VMEM((8, KP), jnp.int32),
        pltpu.VMEM((464,), jnp.float32),
        pltpu.SemaphoreType.DMA,
    ],
)
def _osc_gather(tab_in, osc_out, tab, sco_o, sem1):
    b = lax.axis_index("s")
    half = lax.axis_index("c")
    start = half * HW

    pltpu.async_copy(tab_in.at[b], tab, sem1).wait()

    for i in range(CHUNKS):
        pos = i * 16
        ov = jnp.minimum(jnp.maximum(tab[R_O, pl.ds(start + pos, 16)], 0), K - 1)
        raw = plsc.load_gather(tab, [jnp.zeros((16,), jnp.int32) + R_SC, ov])
        sco_o[pl.ds(pos, 16)] = plsc.bitcast(raw, jnp.float32)

    pltpu.async_copy(
        sco_o.at[pl.ds(0, HW)], osc_out.at[b, pl.ds(start, HW)], sem1
    ).wait()


@functools.partial(
    pl.kernel,
    out_type=(
        jax.ShapeDtypeStruct((B, PW), jnp.int32),
        jax.ShapeDtypeStruct((B, 4, 2, PW), jnp.float32),
    ),
    mesh=_mesh,
    compiler_params=_sc_params,
    scratch_types=[
        pltpu.VMEM((8, KP), jnp.int32),
        pltpu.VMEM((2, 16), jnp.int32),
        pltpu.VMEM((464,), jnp.int32),
        pltpu.VMEM((4, 2, 464), jnp.float32),
        pltpu.SemaphoreType.DMA,
        pltpu.SemaphoreType.DMA,
        pltpu.SemaphoreType.DMA,
    ],
)
def _pair_gather(tab_in, tsizes, lab_out, box_out,
                 tab, tsb, lab_o, bo, sem1, sem2, sem3):
    b = lax.axis_index("s")
    half = lax.axis_index("c")
    start = half * HW

    c1 = pltpu.async_copy(tab_in.at[b], tab, sem1)
    c2 = pltpu.async_copy(tsizes, tsb, sem2)
    c1.wait()
    c2.wait()

    bsplat = jnp.zeros((16,), jnp.int32) + b
    sh = plsc.load_gather(tsb, [jnp.zeros((16,), jnp.int32), bsplat])
    sw = plsc.load_gather(tsb, [jnp.zeros((16,), jnp.int32) + 1, bsplat])
    sh = sh.astype(jnp.float32)
    sw = sw.astype(jnp.float32)

    zeros16 = jnp.zeros((16,), jnp.int32)

    for i in range(CHUNKS):
        pos = i * 16
        hv = jnp.minimum(jnp.maximum(tab[R_H, pl.ds(start + pos, 16)], 0), K - 1)
        ov = jnp.minimum(jnp.maximum(tab[R_O, pl.ds(start + pos, 16)], 0), K - 1)

        lab_o[pl.ds(pos, 16)] = plsc.load_gather(tab, [zeros16 + R_LB, ov])

        for t, idxv in ((0, hv), (1, ov)):
            def _bx(row, idxv=idxv):
                raw = plsc.load_gather(tab, [zeros16 + row, idxv])
                return plsc.bitcast(raw, jnp.float32)

            cx = _bx(R_CX)
            cy = _bx(R_CY)
            hw2 = _bx(R_W) * 0.5
            hh2 = _bx(R_HB) * 0.5
            bo[0, t, pl.ds(pos, 16)] = (cx - hw2) * sw
            bo[1, t, pl.ds(pos, 16)] = (cy - hh2) * sh
            bo[2, t, pl.ds(pos, 16)] = (cx + hw2) * sw
            bo[3, t, pl.ds(pos, 16)] = (cy + hh2) * sh

    sems = (sem1, sem2, sem3)
    outs = [
        pltpu.async_copy(
            lab_o.at[pl.ds(0, HW)], lab_out.at[b, pl.ds(start, HW)], sem1
        ),
    ]
    for t in range(2):
        for c in range(4):
            outs.append(
                pltpu.async_copy(
                    bo.at[c, t, pl.ds(0, HW)],
                    box_out.at[b, c, t, pl.ds(start, HW)],
                    sems[(4 * t + c) % 3],
                )
            )
    for o in outs:
        o.wait()


def _verb_body(act_ref, osc_ref, out_ref):
    sig = 1.0 / (1.0 + jnp.exp(-act_ref[...]))
    s = osc_ref[0]
    for j in range(PT):
        out_ref[j] = sig[j] * jnp.broadcast_to(s[:, j : j + 1], (B, A))


_verb = pl.pallas_call(
    _verb_body,
    grid=(P // PT,),
    in_specs=[
        pl.BlockSpec((PT, B, A), lambda i: (i, 0, 0)),
        pl.BlockSpec((1, B, PT), lambda i: (i, 0, 0)),
    ],
    out_specs=pl.BlockSpec((PT, B, A), lambda i: (i, 0, 0)),
    out_shape=jax.ShapeDtypeStruct((P, B, A), jnp.float32),
)


def kernel(pred_logits, pred_boxes, pred_actions, pred_rel_pairs, target_sizes):
    lgt = jnp.transpose(pred_logits, (2, 0, 1))
    act = jnp.transpose(pred_actions, (1, 0, 2))
    hido = jnp.transpose(pred_rel_pairs, (0, 2, 1))
    pbox = jnp.transpose(pred_boxes, (0, 2, 1))
    ts = jnp.transpose(target_sizes, (1, 0))

    tab = _score_label(lgt, hido, pbox)

    osc = _osc_gather(tab)
    osc_r = jnp.transpose(osc[:, :P].reshape(B, P // PT, PT), (1, 0, 2))
    verb_t = _verb(act, osc_r)

    lab1, box4 = _pair_gather(tab, ts)

    labels = jnp.concatenate(
        [jnp.zeros((B, P), jnp.int32), lab1[:, :P]], axis=1
    )
    b_out = jnp.transpose(box4[:, :, :, :P].reshape(B, 4, 2 * P), (0, 2, 1))
    verb = jnp.transpose(verb_t, (1, 0, 2))
    return labels, b_out, verb


# speedup vs baseline: 1.0887x; 1.0887x over previous
import functools

import jax
import jax.numpy as jnp
from jax import lax
from jax.experimental import pallas as pl
from jax.experimental.pallas import tpu as pltpu
from jax.experimental.pallas import tpu_sc as plsc

B, K, P, C, A = 16, 900, 900, 92, 117
NCLS = 80
KP = 904
HW = 456
PW = 2 * HW
CHUNKS = 29
BB = 8
PT = 300

R_H, R_O, R_CX, R_CY, R_W, R_HB, R_SC, R_LB = range(8)


def _score_label_body(lgt_ref, hido_ref, pbox_ref, out_ref):
    x = lgt_ref[...]
    cl = lax.broadcasted_iota(jnp.int32, (C, BB, K), 0)
    neg = jnp.float32(-jnp.inf)
    x80 = jnp.where(cl < NCLS, x, neg)
    m80 = jnp.max(x80, axis=0, keepdims=True)
    lab = jnp.min(jnp.where(x80 == m80, cl, C), axis=0)
    x81 = jnp.where(cl < NCLS + 1, x, neg)
    m81 = jnp.max(x81, axis=0, keepdims=True)
    se = jnp.sum(jnp.exp(x81 - m81), axis=0)
    score = jnp.exp(m80[0] - m81[0]) / se
    out_ref[:, pl.ds(0, 2), pl.ds(0, K)] = hido_ref[...]
    out_ref[:, pl.ds(2, 4), pl.ds(0, K)] = lax.bitcast_convert_type(
        pbox_ref[...], jnp.int32
    )
    out_ref[:, pl.ds(R_SC, 1), pl.ds(0, K)] = lax.bitcast_convert_type(
        score, jnp.int32
    )[:, None]
    out_ref[:, pl.ds(R_LB, 1), pl.ds(0, K)] = lab[:, None]


_score_label = pl.pallas_call(
    _score_label_body,
    grid=(B // BB,),
    in_specs=[
        pl.BlockSpec((C, BB, K), lambda i: (0, i, 0)),
        pl.BlockSpec((BB, 2, K), lambda i: (i, 0, 0)),
        pl.BlockSpec((BB, 4, K), lambda i: (i, 0, 0)),
    ],
    out_specs=pl.BlockSpec((BB, 8, KP), lambda i: (i, 0, 0)),
    out_shape=jax.ShapeDtypeStruct((B, 8, KP), jnp.int32),
)


_mesh = plsc.VectorSubcoreMesh(core_axis_name="c", subcore_axis_name="s")
_sc_params = pltpu.CompilerParams(
    needs_layout_passes=False, use_tc_tiling_on_sc=False
)


@functools.partial(
    pl.kernel,
    out_type=(
        jax.ShapeDtypeStruct((B, PW), jnp.float32),
        jax.ShapeDtypeStruct((B, PW), jnp.int32),
        jax.ShapeDtypeStruct((B, 4, 2, PW), jnp.float32),
    ),
    mesh=_mesh,
    compiler_params=_sc_params,
    scratch_types=[
        pltpu.VMEM((8, KP), jnp.int32),
        pltpu.VMEM((2, 16), jnp.int32),
        pltpu.VMEM((464,), jnp.float32),
        pltpu.VMEM((464,), jnp.int32),
        pltpu.VMEM((4, 2, 464), jnp.float32),
        pltpu.SemaphoreType.DMA,
        pltpu.SemaphoreType.DMA,
        pltpu.SemaphoreType.DMA,
    ],
)
def _pair_gather(tab_in, tsizes, osc_out, lab_out, box_out,
                 tab, tsb, sco_o, lab_o, bo, sem1, sem2, sem3):
    b = lax.axis_index("s")
    half = lax.axis_index("c")
    start = half * HW

    c1 = pltpu.async_copy(tab_in.at[b], tab, sem1)
    c2 = pltpu.async_copy(tsizes, tsb, sem2)
    c1.wait()
    c2.wait()

    bsplat = jnp.zeros((16,), jnp.int32) + b
    sh = plsc.load_gather(tsb, [jnp.zeros((16,), jnp.int32), bsplat])
    sw = plsc.load_gather(tsb, [jnp.zeros((16,), jnp.int32) + 1, bsplat])
    sh = sh.astype(jnp.float32)
    sw = sw.astype(jnp.float32)

    zeros16 = jnp.zeros((16,), jnp.int32)

    for i in range(CHUNKS):
        pos = i * 16
        hv = jnp.minimum(jnp.maximum(tab[R_H, pl.ds(start + pos, 16)], 0), K - 1)
        ov = jnp.minimum(jnp.maximum(tab[R_O, pl.ds(start + pos, 16)], 0), K - 1)

        lab_o[pl.ds(pos, 16)] = plsc.load_gather(tab, [zeros16 + R_LB, ov])
        sraw = plsc.load_gather(tab, [zeros16 + R_SC, ov])
        sco_o[pl.ds(pos, 16)] = plsc.bitcast(sraw, jnp.float32)

        for t, idxv in ((0, hv), (1, ov)):
            def _bx(row, idxv=idxv):
                raw = plsc.load_gather(tab, [zeros16 + row, idxv])
                return plsc.bitcast(raw, jnp.float32)

            cx = _bx(R_CX)
            cy = _bx(R_CY)
            hw2 = _bx(R_W) * 0.5
            hh2 = _bx(R_HB) * 0.5
            bo[0, t, pl.ds(pos, 16)] = (cx - hw2) * sw
            bo[1, t, pl.ds(pos, 16)] = (cy - hh2) * sh
            bo[2, t, pl.ds(pos, 16)] = (cx + hw2) * sw
            bo[3, t, pl.ds(pos, 16)] = (cy + hh2) * sh

    sems = (sem1, sem2, sem3)
    outs = [
        pltpu.async_copy(
            sco_o.at[pl.ds(0, HW)], osc_out.at[b, pl.ds(start, HW)], sem1
        ),
        pltpu.async_copy(
            lab_o.at[pl.ds(0, HW)], lab_out.at[b, pl.ds(start, HW)], sem2
        ),
    ]
    for t in range(2):
        for c in range(4):
            outs.append(
                pltpu.async_copy(
                    bo.at[c, t, pl.ds(0, HW)],
                    box_out.at[b, c, t, pl.ds(start, HW)],
                    sems[(4 * t + c) % 3],
                )
            )
    for o in outs:
        o.wait()


def _verb_body(act_ref, osc_ref, out_ref):
    sig = 1.0 / (1.0 + jnp.exp(-act_ref[...]))
    s = osc_ref[0]
    for j in range(PT):
        out_ref[j] = sig[j] * jnp.broadcast_to(s[:, j : j + 1], (B, A))


_verb = pl.pallas_call(
    _verb_body,
    grid=(P // PT,),
    in_specs=[
        pl.BlockSpec((PT, B, A), lambda i: (i, 0, 0)),
        pl.BlockSpec((1, B, PT), lambda i: (i, 0, 0)),
    ],
    out_specs=pl.BlockSpec((PT, B, A), lambda i: (i, 0, 0)),
    out_shape=jax.ShapeDtypeStruct((P, B, A), jnp.float32),
)


def kernel(pred_logits, pred_boxes, pred_actions, pred_rel_pairs, target_sizes):
    lgt = jnp.transpose(pred_logits, (2, 0, 1))
    act = jnp.transpose(pred_actions, (1, 0, 2))
    hido = jnp.transpose(pred_rel_pairs, (0, 2, 1))
    pbox = jnp.transpose(pred_boxes, (0, 2, 1))
    ts = jnp.transpose(target_sizes, (1, 0))

    tab = _score_label(lgt, hido, pbox)

    osc, lab1, box4 = _pair_gather(tab, ts)
    osc_r = jnp.transpose(osc[:, :P].reshape(B, P // PT, PT), (1, 0, 2))
    verb_t = _verb(act, osc_r)

    labels = jnp.concatenate(
        [jnp.zeros((B, P), jnp.int32), lab1[:, :P]], axis=1
    )
    b_out = jnp.transpose(box4[:, :, :, :P].reshape(B, 4, 2 * P), (0, 2, 1))
    verb = jnp.transpose(verb_t, (1, 0, 2))
    return labels, b_out, verb


# speedup vs baseline: 1.1199x; 1.0287x over previous
import functools

import jax
import jax.numpy as jnp
from jax import lax
from jax.experimental import pallas as pl
from jax.experimental.pallas import tpu as pltpu
from jax.experimental.pallas import tpu_sc as plsc

B, K, P, C, A = 16, 900, 900, 92, 117
NCLS = 80
KP = 904
HW = 456
PW = 2 * HW
CHUNKS = 29
BB = 8
PT = 450

R_H, R_O, R_CX, R_CY, R_W, R_HB, R_SC, R_LB = range(8)


def _score_label_body(lgt_ref, hido_ref, pbox_ref, out_ref):
    x = lgt_ref[...]
    cl = lax.broadcasted_iota(jnp.int32, (C, BB, K), 0)
    neg = jnp.float32(-jnp.inf)
    x80 = jnp.where(cl < NCLS, x, neg)
    m80 = jnp.max(x80, axis=0, keepdims=True)
    lab = jnp.min(jnp.where(x80 == m80, cl, C), axis=0)
    x81 = jnp.where(cl < NCLS + 1, x, neg)
    m81 = jnp.max(x81, axis=0, keepdims=True)
    se = jnp.sum(jnp.exp(x81 - m81), axis=0)
    score = jnp.exp(m80[0] - m81[0]) / se
    out_ref[:, pl.ds(0, 2), pl.ds(0, K)] = hido_ref[...]
    out_ref[:, pl.ds(2, 4), pl.ds(0, K)] = lax.bitcast_convert_type(
        pbox_ref[...], jnp.int32
    )
    out_ref[:, pl.ds(R_SC, 1), pl.ds(0, K)] = lax.bitcast_convert_type(
        score, jnp.int32
    )[:, None]
    out_ref[:, pl.ds(R_LB, 1), pl.ds(0, K)] = lab[:, None]


_score_label = pl.pallas_call(
    _score_label_body,
    grid=(B // BB,),
    in_specs=[
        pl.BlockSpec((C, BB, K), lambda i: (0, i, 0)),
        pl.BlockSpec((BB, 2, K), lambda i: (i, 0, 0)),
        pl.BlockSpec((BB, 4, K), lambda i: (i, 0, 0)),
    ],
    out_specs=pl.BlockSpec((BB, 8, KP), lambda i: (i, 0, 0)),
    out_shape=jax.ShapeDtypeStruct((B, 8, KP), jnp.int32),
)


_mesh = plsc.VectorSubcoreMesh(core_axis_name="c", subcore_axis_name="s")
_sc_params = pltpu.CompilerParams(
    needs_layout_passes=False, use_tc_tiling_on_sc=False
)


@functools.partial(
    pl.kernel,
    out_type=(
        jax.ShapeDtypeStruct((B, PW), jnp.float32),
        jax.ShapeDtypeStruct((B, PW), jnp.int32),
        jax.ShapeDtypeStruct((B, 4, 2, PW), jnp.float32),
    ),
    mesh=_mesh,
    compiler_params=_sc_params,
    scratch_types=[
        pltpu.VMEM((8, KP), jnp.int32),
        pltpu.VMEM((2, 16), jnp.int32),
        pltpu.VMEM((464,), jnp.float32),
        pltpu.VMEM((464,), jnp.int32),
        pltpu.VMEM((4, 2, 464), jnp.float32),
        pltpu.SemaphoreType.DMA,
        pltpu.SemaphoreType.DMA,
        pltpu.SemaphoreType.DMA,
    ],
)
def _pair_gather(tab_in, tsizes, osc_out, lab_out, box_out,
                 tab, tsb, sco_o, lab_o, bo, sem1, sem2, sem3):
    b = lax.axis_index("s")
    half = lax.axis_index("c")
    start = half * HW

    c1 = pltpu.async_copy(tab_in.at[b], tab, sem1)
    c2 = pltpu.async_copy(tsizes, tsb, sem2)
    c1.wait()
    c2.wait()

    bsplat = jnp.zeros((16,), jnp.int32) + b
    sh = plsc.load_gather(tsb, [jnp.zeros((16,), jnp.int32), bsplat])
    sw = plsc.load_gather(tsb, [jnp.zeros((16,), jnp.int32) + 1, bsplat])
    sh = sh.astype(jnp.float32)
    sw = sw.astype(jnp.float32)

    zeros16 = jnp.zeros((16,), jnp.int32)

    for i in range(CHUNKS):
        pos = i * 16
        hv = jnp.minimum(jnp.maximum(tab[R_H, pl.ds(start + pos, 16)], 0), K - 1)
        ov = jnp.minimum(jnp.maximum(tab[R_O, pl.ds(start + pos, 16)], 0), K - 1)

        lab_o[pl.ds(pos, 16)] = plsc.load_gather(tab, [zeros16 + R_LB, ov])
        sraw = plsc.load_gather(tab, [zeros16 + R_SC, ov])
        sco_o[pl.ds(pos, 16)] = plsc.bitcast(sraw, jnp.float32)

        for t, idxv in ((0, hv), (1, ov)):
            def _bx(row, idxv=idxv):
                raw = plsc.load_gather(tab, [zeros16 + row, idxv])
                return plsc.bitcast(raw, jnp.float32)

            cx = _bx(R_CX)
            cy = _bx(R_CY)
            hw2 = _bx(R_W) * 0.5
            hh2 = _bx(R_HB) * 0.5
            bo[0, t, pl.ds(pos, 16)] = (cx - hw2) * sw
            bo[1, t, pl.ds(pos, 16)] = (cy - hh2) * sh
            bo[2, t, pl.ds(pos, 16)] = (cx + hw2) * sw
            bo[3, t, pl.ds(pos, 16)] = (cy + hh2) * sh

    sems = (sem1, sem2, sem3)
    outs = [
        pltpu.async_copy(
            sco_o.at[pl.ds(0, HW)], osc_out.at[b, pl.ds(start, HW)], sem1
        ),
        pltpu.async_copy(
            lab_o.at[pl.ds(0, HW)], lab_out.at[b, pl.ds(start, HW)], sem2
        ),
    ]
    for t in range(2):
        for c in range(4):
            outs.append(
                pltpu.async_copy(
                    bo.at[c, t, pl.ds(0, HW)],
                    box_out.at[b, c, t, pl.ds(start, HW)],
                    sems[(4 * t + c) % 3],
                )
            )
    for o in outs:
        o.wait()


def _verb_body(act_ref, osc_ref, out_ref):
    sig = 1.0 / (1.0 + jnp.exp(-act_ref[...]))
    s = osc_ref[0]
    for j in range(PT):
        out_ref[j] = sig[j] * jnp.broadcast_to(s[:, j : j + 1], (B, A))


_verb = pl.pallas_call(
    _verb_body,
    grid=(P // PT,),
    in_specs=[
        pl.BlockSpec((PT, B, A), lambda i: (i, 0, 0)),
        pl.BlockSpec((1, B, PT), lambda i: (i, 0, 0)),
    ],
    out_specs=pl.BlockSpec((PT, B, A), lambda i: (i, 0, 0)),
    out_shape=jax.ShapeDtypeStruct((P, B, A), jnp.float32),
)


def kernel(pred_logits, pred_boxes, pred_actions, pred_rel_pairs, target_sizes):
    lgt = jnp.transpose(pred_logits, (2, 0, 1))
    act = jnp.transpose(pred_actions, (1, 0, 2))
    hido = jnp.transpose(pred_rel_pairs, (0, 2, 1))
    pbox = jnp.transpose(pred_boxes, (0, 2, 1))
    ts = jnp.transpose(target_sizes, (1, 0))

    tab = _score_label(lgt, hido, pbox)

    osc, lab1, box4 = _pair_gather(tab, ts)
    osc_r = jnp.transpose(osc[:, :P].reshape(B, P // PT, PT), (1, 0, 2))
    verb_t = _verb(act, osc_r)

    labels = jnp.concatenate(
        [jnp.zeros((B, P), jnp.int32), lab1[:, :P]], axis=1
    )
    b_out = jnp.transpose(box4[:, :, :, :P].reshape(B, 4, 2 * P), (0, 2, 1))
    verb = jnp.transpose(verb_t, (1, 0, 2))
    return labels, b_out, verb
